# baseline (speedup 1.0000x reference)
import functools

import jax
import jax.numpy as jnp
from jax import lax
from jax.experimental import pallas as pl
from jax.experimental.pallas import tpu as pltpu
from jax.experimental.pallas import tpu_sc as plsc

_NUM_GROUPS = 16
_FEAT = 256
_FH = 64
_FW = 64
_HW = _FH * _FW
_ROWS = _NUM_GROUPS * _HW

_NC = 2
_NS = 16
_NW = _NC * _NS
_CHUNK = 128


def _transpose_body(in_ref, out_ref):
    out_ref[0] = in_ref[0].T


def _build_table(feat_map):
    fm3 = feat_map.reshape(_NUM_GROUPS, _FEAT, _HW)
    t = pl.pallas_call(
        _transpose_body,
        grid=(_NUM_GROUPS, 8),
        in_specs=[pl.BlockSpec((1, _FEAT, _HW // 8), lambda g, j: (g, 0, j))],
        out_specs=pl.BlockSpec((1, _HW // 8, _FEAT), lambda g, j: (g, j, 0)),
        out_shape=jax.ShapeDtypeStruct((_NUM_GROUPS, _HW, _FEAT), jnp.float32),
    )(fm3)
    return t.reshape(_ROWS, _FEAT)


def _gather(table, grp, grid, n_pad):
    chunks_per_w = n_pad // (_NW * _CHUNK)
    rows_per_w = chunks_per_w * _CHUNK
    mesh = plsc.VectorSubcoreMesh(core_axis_name="c", subcore_axis_name="s")

    @functools.partial(
        pl.kernel,
        mesh=mesh,
        compiler_params=pltpu.CompilerParams(needs_layout_passes=False),
        out_type=jax.ShapeDtypeStruct((n_pad, _FEAT), jnp.float32),
        scratch_types=[
            pltpu.VMEM((_CHUNK,), jnp.int32),
            pltpu.VMEM((2 * _CHUNK,), jnp.int32),
            pltpu.VMEM((_CHUNK,), jnp.int32),
            pltpu.VMEM((_CHUNK, _FEAT), jnp.float32),
            pltpu.SemaphoreType.DMA,
        ],
    )
    def k(table_hbm, grp_hbm, grid_hbm, out_hbm, grp_v, grid_v, idx_v, rows_v, sem):
        wid = lax.axis_index("s") * _NC + lax.axis_index("c")
        w_base = wid * rows_per_w

        def body(j, carry):
            base = w_base + j * _CHUNK
            pltpu.sync_copy(grp_hbm.at[pl.ds(base, _CHUNK)], grp_v)
            pltpu.sync_copy(grid_hbm.at[pl.ds(2 * base, 2 * _CHUNK)], grid_v)
            for i in range(_CHUNK // 16):
                pairs = (lax.iota(jnp.int32, 16) + jnp.int32(i * 16)) * 2
                g = grp_v[pl.ds(i * 16, 16)]
                x = plsc.load_gather(grid_v, [pairs])
                y = plsc.load_gather(grid_v, [pairs + 1])
                idx_v[pl.ds(i * 16, 16)] = g * _HW + y * _FW + x
            pltpu.async_copy(table_hbm.at[idx_v], rows_v, sem).wait()
            pltpu.sync_copy(rows_v, out_hbm.at[pl.ds(base, _CHUNK)])
            return carry

        lax.fori_loop(0, chunks_per_w, body, 0)

    return k(table, grp, grid)


def kernel(feat_map, grp_ids, grid_ids):
    n = grp_ids.shape[0]
    grp = grp_ids.astype(jnp.int32)
    grid = grid_ids.astype(jnp.int32)
    per = _NW * _CHUNK
    n_pad = ((n + per - 1) // per) * per
    pad = n_pad - n
    if pad:
        grp = jnp.pad(grp, (0, pad))
        grid = jnp.pad(grid, ((0, pad), (0, 0)))
    grid = grid.reshape(-1)
    table = _build_table(feat_map)
    out = _gather(table, grp, grid, n_pad)
    return out[:n]
